# baseline (speedup 1.0000x reference)
import jax
import jax.numpy as jnp
from jax.experimental import pallas as pl
from jax.experimental.pallas import tpu as pltpu

N = 10000
NFEAT = 128
NHID = 128

BM = 400
BH = BM // 2


def _main_kernel(adj_a_ref, adj_b_ref, xb_ref, x_ref, w1_ref, w2_ref, b_ref,
                 out_ref):
    x = x_ref[...]
    sa = jnp.dot(adj_a_ref[...].astype(jnp.bfloat16), x,
                 preferred_element_type=jnp.float32)
    sb = jnp.dot(adj_b_ref[...].astype(jnp.bfloat16), x,
                 preferred_element_type=jnp.float32)
    s = jnp.concatenate([sa, sb], axis=0)
    h = jnp.dot(s.astype(jnp.bfloat16), w2_ref[...],
                preferred_element_type=jnp.float32)
    z = jnp.dot(xb_ref[...], w1_ref[...], preferred_element_type=jnp.float32)
    out_ref[...] = jnp.maximum(h + z + b_ref[...], 0.0)


def kernel(x, adj, W, b):
    W1 = W[:NFEAT].astype(jnp.bfloat16)
    W2 = W[NFEAT:].astype(jnp.bfloat16)
    x_bf = x.astype(jnp.bfloat16)
    b2d = b.reshape(1, NHID)

    out = pl.pallas_call(
        _main_kernel,
        grid=(N // BM,),
        in_specs=[
            pl.BlockSpec((BH, N), lambda i: (2 * i, 0)),
            pl.BlockSpec((BH, N), lambda i: (2 * i + 1, 0)),
            pl.BlockSpec((BM, NFEAT), lambda i: (i, 0)),
            pl.BlockSpec((N, NFEAT), lambda i: (0, 0)),
            pl.BlockSpec((NFEAT, NHID), lambda i: (0, 0)),
            pl.BlockSpec((NFEAT, NHID), lambda i: (0, 0)),
            pl.BlockSpec((1, NHID), lambda i: (0, 0)),
        ],
        out_specs=pl.BlockSpec((BM, NHID), lambda i: (i, 0)),
        out_shape=jax.ShapeDtypeStruct((N, NHID), jnp.float32),
        compiler_params=pltpu.CompilerParams(
            dimension_semantics=("parallel",)),
    )(adj, adj, x_bf, x_bf, W1, W2, b2d)
    return out


# speedup vs baseline: 1.0216x; 1.0216x over previous
import jax
import jax.numpy as jnp
from jax.experimental import pallas as pl
from jax.experimental.pallas import tpu as pltpu

N = 10000
NFEAT = 128
NHID = 128

BM = 400


def _main_kernel(adj_ref, x_ref, w1_ref, w2_ref, b_ref, out_ref):
    i = pl.program_id(0)
    a = adj_ref[...].astype(jnp.bfloat16)
    s = jnp.dot(a, x_ref[...], preferred_element_type=jnp.float32)
    h = jnp.dot(s.astype(jnp.bfloat16), w2_ref[...],
                preferred_element_type=jnp.float32)
    xb = x_ref[pl.ds(i * BM, BM), :]
    z = jnp.dot(xb, w1_ref[...], preferred_element_type=jnp.float32)
    out_ref[...] = jnp.maximum(h + z + b_ref[...], 0.0)


def kernel(x, adj, W, b):
    W1 = W[:NFEAT].astype(jnp.bfloat16)
    W2 = W[NFEAT:].astype(jnp.bfloat16)
    x_bf = x.astype(jnp.bfloat16)
    b2d = b.reshape(1, NHID)

    out = pl.pallas_call(
        _main_kernel,
        grid=(N // BM,),
        in_specs=[
            pl.BlockSpec((BM, N), lambda i: (i, 0)),
            pl.BlockSpec((N, NFEAT), lambda i: (0, 0)),
            pl.BlockSpec((NFEAT, NHID), lambda i: (0, 0)),
            pl.BlockSpec((NFEAT, NHID), lambda i: (0, 0)),
            pl.BlockSpec((1, NHID), lambda i: (0, 0)),
        ],
        out_specs=pl.BlockSpec((BM, NHID), lambda i: (i, 0)),
        out_shape=jax.ShapeDtypeStruct((N, NHID), jnp.float32),
        compiler_params=pltpu.CompilerParams(
            dimension_semantics=("parallel",)),
    )(adj, x_bf, W1, W2, b2d)
    return out


# speedup vs baseline: 1.0223x; 1.0007x over previous
import jax
import jax.numpy as jnp
from jax.experimental import pallas as pl
from jax.experimental.pallas import tpu as pltpu

N = 10000
NFEAT = 128
NHID = 128

BM = 400


def _main_kernel(adj_ref, x_ref, w1_ref, w2_ref, b_ref, out_ref):
    i = pl.program_id(0)
    a = adj_ref[...].astype(jnp.bfloat16)
    s = jnp.dot(a, x_ref[...], preferred_element_type=jnp.float32)
    h = jnp.dot(s.astype(jnp.bfloat16), w2_ref[...],
                preferred_element_type=jnp.float32)
    xb = x_ref[pl.ds(i * BM, BM), :]
    z = jnp.dot(xb, w1_ref[...], preferred_element_type=jnp.float32)
    out_ref[...] = jnp.maximum(h + z + b_ref[...], 0.0)


def kernel(x, adj, W, b):
    W1 = W[:NFEAT].astype(jnp.bfloat16)
    W2 = W[NFEAT:].astype(jnp.bfloat16)
    x_bf = x.astype(jnp.bfloat16)
    b2d = b.reshape(1, NHID)

    out = pl.pallas_call(
        _main_kernel,
        grid=(N // BM,),
        in_specs=[
            pl.BlockSpec((BM, N), lambda i: (i, 0)),
            pl.BlockSpec((N, NFEAT), lambda i: (0, 0)),
            pl.BlockSpec((NFEAT, NHID), lambda i: (0, 0)),
            pl.BlockSpec((NFEAT, NHID), lambda i: (0, 0)),
            pl.BlockSpec((1, NHID), lambda i: (0, 0)),
        ],
        out_specs=pl.BlockSpec((BM, NHID), lambda i: (i, 0)),
        out_shape=jax.ShapeDtypeStruct((N, NHID), jnp.float32),
        compiler_params=pltpu.CompilerParams(
            dimension_semantics=("parallel",),
            allow_input_fusion=[False, True, True, True, True]),
    )(adj, x_bf, W1, W2, b2d)
    return out
